# baseline (speedup 1.0000x reference)
import functools

import jax
import jax.numpy as jnp
from jax import lax
from jax.experimental import pallas as pl
from jax.experimental.pallas import tpu as pltpu
from jax.experimental.pallas import tpu_sc as plsc

NC = 2
NS = 16
NW = NC * NS


@functools.partial(jax.jit, static_argnums=(2, 3))
def _sc_embed(ids, table, chunk, nbuf):
    B, S = ids.shape
    D = table.shape[1]
    n_per_w = (B * S) // NW
    wpb = S // n_per_w
    n_chunks = n_per_w // chunk
    mesh = plsc.VectorSubcoreMesh(
        core_axis_name="c", subcore_axis_name="s",
        num_cores=NC, num_subcores=NS)

    @functools.partial(
        pl.kernel,
        out_type=jax.ShapeDtypeStruct((B, S, D), jnp.float32),
        mesh=mesh,
        scratch_types=[
            pltpu.VMEM((n_per_w,), jnp.int32),
            pltpu.VMEM((nbuf, chunk, D), jnp.float32),
            [pltpu.SemaphoreType.DMA] * nbuf,
            [pltpu.SemaphoreType.DMA] * nbuf,
        ],
    )
    def k(idx_hbm, table_hbm, out_hbm, idx_v, rows_v, gsems, ssems):
        wid = lax.axis_index("s") * NC + lax.axis_index("c")
        b = wid // wpb
        col = (wid % wpb) * n_per_w

        pltpu.sync_copy(idx_hbm.at[b, pl.ds(col, n_per_w)], idx_v)

        def gather(c):
            bf = c % nbuf
            return pltpu.async_copy(
                table_hbm.at[idx_v.at[pl.ds(c * chunk, chunk)]],
                rows_v.at[bf], gsems[bf])

        def scatter(c):
            bf = c % nbuf
            return pltpu.async_copy(
                rows_v.at[bf], out_hbm.at[b, pl.ds(col + c * chunk, chunk)],
                ssems[bf])

        depth = min(3, nbuf - 1)
        gd = [None] * n_chunks
        sd = [None] * n_chunks
        for c in range(min(depth, n_chunks)):
            gd[c] = gather(c)
        for c in range(n_chunks):
            gd[c].wait()
            sd[c] = scatter(c)
            nxt = c + depth
            if nxt < n_chunks:
                if nxt >= nbuf:
                    sd[nxt - nbuf].wait()
                gd[nxt] = gather(nxt)
        for c in range(max(0, n_chunks - nbuf), n_chunks):
            sd[c].wait()

    return k(ids, table)


def kernel(inputs_id, embed_tokens_weight):
    return _sc_embed(inputs_id.astype(jnp.int32), embed_tokens_weight, 16, 6)


# speedup vs baseline: 1.0347x; 1.0347x over previous
import functools

import jax
import jax.numpy as jnp
from jax import lax
from jax.experimental import pallas as pl
from jax.experimental.pallas import tpu as pltpu
from jax.experimental.pallas import tpu_sc as plsc

NC = 2
NS = 16
NW = NC * NS


@functools.partial(jax.jit, static_argnums=(2, 3))
def _sc_embed(ids, table, chunk, nbuf):
    B, S = ids.shape
    D = table.shape[1]
    n_per_w = (B * S) // NW
    wpb = S // n_per_w
    n_chunks = n_per_w // chunk
    mesh = plsc.VectorSubcoreMesh(
        core_axis_name="c", subcore_axis_name="s",
        num_cores=NC, num_subcores=NS)

    @functools.partial(
        pl.kernel,
        out_type=jax.ShapeDtypeStruct((B, S, D), jnp.float32),
        mesh=mesh,
        scratch_types=[
            pltpu.VMEM((n_per_w,), jnp.int32),
            pltpu.VMEM((nbuf, chunk, D), jnp.float32),
            [pltpu.SemaphoreType.DMA] * nbuf,
            [pltpu.SemaphoreType.DMA] * nbuf,
        ],
    )
    def k(idx_hbm, table_hbm, out_hbm, idx_v, rows_v, gsems, ssems):
        wid = lax.axis_index("s") * NC + lax.axis_index("c")
        b = wid // wpb
        col = (wid % wpb) * n_per_w

        pltpu.sync_copy(idx_hbm.at[b, pl.ds(col, n_per_w)], idx_v)

        def gather(c):
            bf = c % nbuf
            return pltpu.async_copy(
                table_hbm.at[idx_v.at[pl.ds(c * chunk, chunk)]],
                rows_v.at[bf], gsems[bf])

        def scatter(c):
            bf = c % nbuf
            return pltpu.async_copy(
                rows_v.at[bf], out_hbm.at[b, pl.ds(col + c * chunk, chunk)],
                ssems[bf])

        depth = nbuf - 1
        gd = [None] * n_chunks
        sd = [None] * n_chunks
        for c in range(min(depth, n_chunks)):
            gd[c] = gather(c)
        for c in range(n_chunks):
            gd[c].wait()
            sd[c] = scatter(c)
            nxt = c + depth
            if nxt < n_chunks:
                if nxt >= nbuf:
                    sd[nxt - nbuf].wait()
                gd[nxt] = gather(nxt)
        for c in range(max(0, n_chunks - nbuf), n_chunks):
            sd[c].wait()

    return k(ids, table)


def kernel(inputs_id, embed_tokens_weight):
    return _sc_embed(inputs_id.astype(jnp.int32), embed_tokens_weight, 16, 6)
